# Initial kernel scaffold; baseline (speedup 1.0000x reference)
import jax
import jax.numpy as jnp
from jax.experimental import pallas as pl


def kernel(node_feat, edge_index, inc_vals, node_degs, edge_degs, Wn, bn, We, be, W1, b1, W2, b2, Wout, bout):
    raise NotImplementedError("write your pallas kernel here")


# speedup vs baseline: 17.2305x; 17.2305x over previous
import functools

import jax
import jax.numpy as jnp
from jax import lax
from jax.experimental import pallas as pl
from jax.experimental.pallas import tpu as pltpu
from jax.experimental.pallas import tpu_sc as plsc

_N = 10000
_M = 10000
_B = 50
_NG = 200
_MG = 200
_NNZ = 320000
_D = 128
_K = 30
_HALF = (_M * _NG) // 2
_PER_TILE = _NNZ // 16
_FULL_CHUNKS = _PER_TILE // 128
_TAIL = _PER_TILE - _FULL_CHUNKS * 128
_ZB = 10000
_CPT = _HALF // 10

@functools.cache
def _get_build_h():
    mesh = plsc.VectorSubcoreMesh(core_axis_name="c", subcore_axis_name="s")
    return functools.partial(
        pl.kernel,
        out_type=jax.ShapeDtypeStruct((_M * _NG,), jnp.float32),
        mesh=mesh,
        scratch_types=[
            pltpu.VMEM((128,), jnp.int32),
            pltpu.VMEM((128,), jnp.int32),
            pltpu.VMEM((128,), jnp.int32),
            pltpu.VMEM((128,), jnp.float32),
            pltpu.VMEM((_ZB,), jnp.float32),
            pltpu.VMEM_SHARED((_HALF + 16,), jnp.float32),
        ],
    )(_build_h_body)


def _build_h_body(e_hbm, n_hbm, v_hbm, out_hbm, ebuf, nbuf, idxbuf, vbuf, zbuf, acc):
    cid = lax.axis_index("c")
    sid = lax.axis_index("s")
    lo = cid * _HALF

    @pl.when(sid < 10)
    def _zero():
        def zvec(i, carry):
            zbuf[pl.ds(i * 16, 16)] = jnp.zeros((16,), jnp.float32)
            return carry
        lax.fori_loop(0, _ZB // 16, zvec, 0)

        def zcopy(j, carry):
            pltpu.sync_copy(zbuf, acc.at[pl.ds(sid * _CPT + j * _ZB, _ZB)])
            return carry
        lax.fori_loop(0, _CPT // _ZB, zcopy, 0)

    plsc.subcore_barrier()

    base = sid * _PER_TILE

    def do_vec(vi, carry):
        ev = ebuf[pl.ds(vi * 16, 16)]
        nv = nbuf[pl.ds(vi * 16, 16)]
        flat = ev * _NG + lax.rem(nv, _NG)
        loc = flat - lo
        ok = (loc >= 0) & (loc < _HALF)
        idxbuf[pl.ds(vi * 16, 16)] = jnp.where(ok, loc, _HALF)
        return carry

    def chunk(ci, carry):
        off = base + ci * 128
        pltpu.sync_copy(e_hbm.at[pl.ds(off, 128)], ebuf)
        pltpu.sync_copy(n_hbm.at[pl.ds(off, 128)], nbuf)
        pltpu.sync_copy(v_hbm.at[pl.ds(off, 128)], vbuf)
        lax.fori_loop(0, 8, do_vec, 0)
        pltpu.sync_copy(vbuf, acc.at[idxbuf], add=True)
        return carry

    lax.fori_loop(0, _FULL_CHUNKS, chunk, 0)

    toff = base + _FULL_CHUNKS * 128
    pltpu.sync_copy(e_hbm.at[pl.ds(toff, _TAIL)], ebuf.at[pl.ds(0, _TAIL)])
    pltpu.sync_copy(n_hbm.at[pl.ds(toff, _TAIL)], nbuf.at[pl.ds(0, _TAIL)])
    pltpu.sync_copy(v_hbm.at[pl.ds(toff, _TAIL)], vbuf.at[pl.ds(0, _TAIL)])
    for vi in range(_TAIL // 16):
        do_vec(vi, 0)
    for vi in range(_TAIL // 16, 8):
        idxbuf[pl.ds(vi * 16, 16)] = jnp.full((16,), _HALF, jnp.int32)
    pltpu.sync_copy(vbuf, acc.at[idxbuf], add=True)

    plsc.subcore_barrier()

    @pl.when(sid < 10)
    def _copy_out():
        def ccopy(j, carry):
            off = sid * _CPT + j * _ZB
            pltpu.sync_copy(acc.at[pl.ds(off, _ZB)], zbuf)
            pltpu.sync_copy(zbuf, out_hbm.at[pl.ds(lo + off, _ZB)])
            return carry
        lax.fori_loop(0, _CPT // _ZB, ccopy, 0)


def _mm(a, b):
    return lax.dot_general(a, b, (((1,), (0,)), ((), ())),
                           precision=lax.Precision.HIGHEST,
                           preferred_element_type=jnp.float32)


def _mmT(a, b):
    return lax.dot_general(a, b, (((0,), (0,)), ((), ())),
                           precision=lax.Precision.HIGHEST,
                           preferred_element_type=jnp.float32)


def _tc_body(h_ref, nf_ref, nd_ref,
             wn0, wn1, wn2, wn3, bn0, bn1, bn2, bn3,
             we0, we1, we2, we3, be0, be1, be2, be3,
             w1t, b1r, w2r, b2r, wvr, boutr, out_ref):
    H = h_ref[0]
    cur = nf_ref[0]
    ndeg = nd_ref[0]
    Wn = (wn0[...], wn1[...], wn2[...], wn3[...])
    Bn = (bn0[...], bn1[...], bn2[...], bn3[...])
    We = (we0[...], we1[...], we2[...], we3[...])
    Be = (be0[...], be1[...], be2[...], be3[...])

    cats = []
    for lv in range(4):
        P = _mm(cur, Wn[lv])
        n2e = _mm(H, P)
        msg = jnp.tanh(n2e + Bn[lv])
        if lv < 3:
            Q = _mm(msg, We[lv])
        else:
            Q = msg * We[lv][0, 0]
        e2n = _mmT(H, Q)
        cur = jnp.tanh((e2n + Be[lv]) / ndeg)
        cats.append(cur)

    v = cats[3]
    ii = lax.broadcasted_iota(jnp.int32, (_NG, _NG), 0)
    jj = lax.broadcasted_iota(jnp.int32, (_NG, _NG), 1)
    eye = (ii == jj).astype(jnp.float32)
    vrow = _mmT(v, eye)
    before = (vrow > v) | ((vrow == v) & (jj < ii))
    r = jnp.sum(before.astype(jnp.float32), axis=1, keepdims=True)
    ks = lax.broadcasted_iota(jnp.int32, (_NG, _K), 1).astype(jnp.float32)
    O = (r == ks).astype(jnp.float32)

    W1T = w1t[...]
    y1 = (_mm(_mmT(O, cats[0]), W1T[0:32])
          + _mm(_mmT(O, cats[1]), W1T[32:64])
          + _mm(_mmT(O, cats[2]), W1T[64:96])
          + _mmT(O, cats[3]) * W1T[96:97]
          + b1r[...])
    y1 = jnp.maximum(y1, 0.0)

    ri = lax.broadcasted_iota(jnp.int32, (_K // 2, _K), 0)
    ci = lax.broadcasted_iota(jnp.int32, (_K // 2, _K), 1)
    ev = (ci == 2 * ri).astype(jnp.float32)
    od = (ci == 2 * ri + 1).astype(jnp.float32)
    y1p = jnp.maximum(_mm(ev, y1), _mm(od, y1))

    W2r = w2r[...]
    acc = jnp.zeros((11, 32), jnp.float32) + b2r[...]
    for t in range(5):
        acc = acc + _mm(lax.slice(y1p, (t, 0), (t + 11, 16)), W2r[t])
    y2 = jnp.maximum(acc, 0.0)

    Wv = wvr[...]
    o = jnp.zeros((1, 128), jnp.float32) + boutr[...]
    for h in range(11):
        o = o + _mm(lax.slice(y2, (h, 0), (h + 1, 32)),
                    lax.slice(Wv, (h * 32, 0), (h * 32 + 32, 128)))
    out_ref[0] = jnp.maximum(o, 0.0)


def _tc_forward(Hg, nf, nd, Wn, bn, We, be, W1T, b1r, W2r, b2r, Wv, boutr,
                interpret=False):
    def full(x):
        return pl.BlockSpec(x.shape, lambda i: (0,) * x.ndim)

    weights = (*Wn, *bn, *We, *be, W1T, b1r, W2r, b2r, Wv, boutr)
    return pl.pallas_call(
        _tc_body,
        grid=(_B,),
        in_specs=[
            pl.BlockSpec((1, _MG, _NG), lambda i: (i, 0, 0)),
            pl.BlockSpec((1, _NG, _D), lambda i: (i, 0, 0)),
            pl.BlockSpec((1, _NG, 1), lambda i: (i, 0, 0)),
            *[full(w) for w in weights],
        ],
        out_specs=pl.BlockSpec((1, 1, 128), lambda i: (i, 0, 0)),
        out_shape=jax.ShapeDtypeStruct((_B, 1, 128), jnp.float32),
        interpret=interpret,
    )(Hg, nf, nd, *weights).reshape(_B, 128)


def kernel(node_feat, edge_index, inc_vals, node_degs, edge_degs,
           Wn, bn, We, be, W1, b1, W2, b2, Wout, bout):
    e_ids = edge_index[0].astype(jnp.int32)
    n_ids = edge_index[1].astype(jnp.int32)

    hflat = _get_build_h()(e_ids, n_ids, inc_vals.astype(jnp.float32))
    Hg = hflat.reshape(_B, _MG, _NG)

    nf = node_feat.reshape(_B, _NG, _D)
    nd = node_degs.reshape(_B, _NG, 1)
    W1T = W1.T
    W2r = jnp.transpose(W2, (2, 1, 0))
    Wv = jnp.transpose(Wout.reshape(32, 11, 128), (1, 0, 2)).reshape(352, 128)
    bn2 = tuple(b.reshape(1, -1) for b in bn)
    be2 = tuple(b.reshape(1, -1) for b in be)
    return _tc_forward(Hg, nf, nd, Wn, bn2, We, be2,
                       W1T, b1.reshape(1, -1), W2r, b2.reshape(1, -1),
                       Wv, bout.reshape(1, -1))


# speedup vs baseline: 25.5443x; 1.4825x over previous
import functools

import jax
import jax.numpy as jnp
from jax import lax
from jax.experimental import pallas as pl
from jax.experimental.pallas import tpu as pltpu
from jax.experimental.pallas import tpu_sc as plsc

_N = 10000
_M = 10000
_B = 50
_NG = 200
_MG = 200
_NNZ = 320000
_D = 128
_K = 30
_HALF = (_M * _NG) // 2
_PER_TILE = _NNZ // 16
_FULL_CHUNKS = _PER_TILE // 128
_TAIL = _PER_TILE - _FULL_CHUNKS * 128
_ZB = 10000
_CPT = _HALF // 10

@functools.cache
def _get_build_h():
    mesh = plsc.VectorSubcoreMesh(core_axis_name="c", subcore_axis_name="s")
    return functools.partial(
        pl.kernel,
        out_type=jax.ShapeDtypeStruct((_M * _NG,), jnp.float32),
        mesh=mesh,
        scratch_types=[
            pltpu.VMEM((128,), jnp.int32),
            pltpu.VMEM((128,), jnp.int32),
            pltpu.VMEM((128,), jnp.int32),
            pltpu.VMEM((128,), jnp.float32),
            pltpu.VMEM((_ZB,), jnp.float32),
            pltpu.VMEM_SHARED((_HALF + 16,), jnp.float32),
        ],
    )(_build_h_body)


def _build_h_body(e_hbm, n_hbm, v_hbm, out_hbm, ebuf, nbuf, idxbuf, vbuf, zbuf, acc):
    cid = lax.axis_index("c")
    sid = lax.axis_index("s")
    lo = cid * _HALF

    @pl.when(sid < 10)
    def _zero():
        def zvec(i, carry):
            zbuf[pl.ds(i * 16, 16)] = jnp.zeros((16,), jnp.float32)
            return carry
        lax.fori_loop(0, _ZB // 16, zvec, 0)

        def zcopy(j, carry):
            pltpu.sync_copy(zbuf, acc.at[pl.ds(sid * _CPT + j * _ZB, _ZB)])
            return carry
        lax.fori_loop(0, _CPT // _ZB, zcopy, 0)

    plsc.subcore_barrier()

    base = sid * _PER_TILE

    def do_vec(vi, carry):
        ev = ebuf[pl.ds(vi * 16, 16)]
        nv = nbuf[pl.ds(vi * 16, 16)]
        flat = ev * _NG + lax.rem(nv, _NG)
        loc = flat - lo
        ok = (loc >= 0) & (loc < _HALF)
        idxbuf[pl.ds(vi * 16, 16)] = jnp.where(ok, loc, _HALF)
        return carry

    def chunk(ci, carry):
        off = base + ci * 128
        pltpu.sync_copy(e_hbm.at[pl.ds(off, 128)], ebuf)
        pltpu.sync_copy(n_hbm.at[pl.ds(off, 128)], nbuf)
        pltpu.sync_copy(v_hbm.at[pl.ds(off, 128)], vbuf)
        lax.fori_loop(0, 8, do_vec, 0)
        pltpu.sync_copy(vbuf, acc.at[idxbuf], add=True)
        return carry

    lax.fori_loop(0, _FULL_CHUNKS, chunk, 0)

    toff = base + _FULL_CHUNKS * 128
    pltpu.sync_copy(e_hbm.at[pl.ds(toff, _TAIL)], ebuf.at[pl.ds(0, _TAIL)])
    pltpu.sync_copy(n_hbm.at[pl.ds(toff, _TAIL)], nbuf.at[pl.ds(0, _TAIL)])
    pltpu.sync_copy(v_hbm.at[pl.ds(toff, _TAIL)], vbuf.at[pl.ds(0, _TAIL)])
    for vi in range(_TAIL // 16):
        do_vec(vi, 0)
    for vi in range(_TAIL // 16, 8):
        idxbuf[pl.ds(vi * 16, 16)] = jnp.full((16,), _HALF, jnp.int32)
    pltpu.sync_copy(vbuf, acc.at[idxbuf], add=True)

    plsc.subcore_barrier()

    @pl.when(sid < 10)
    def _copy_out():
        def ccopy(j, carry):
            off = sid * _CPT + j * _ZB
            pltpu.sync_copy(acc.at[pl.ds(off, _ZB)], zbuf)
            pltpu.sync_copy(zbuf, out_hbm.at[pl.ds(lo + off, _ZB)])
            return carry
        lax.fori_loop(0, _CPT // _ZB, ccopy, 0)


def _mm(a, b):
    return lax.dot_general(a, b, (((1,), (0,)), ((), ())),
                           preferred_element_type=jnp.float32)


def _mmT(a, b):
    return lax.dot_general(a, b, (((0,), (0,)), ((), ())),
                           preferred_element_type=jnp.float32)


def _tc_body(h_ref, nf_ref, nd_ref,
             wn0, wn1, wn2, wn3, bn0, bn1, bn2, bn3,
             we0, we1, we2, we3, be0, be1, be2, be3,
             w1t, b1r, w2r, b2r, wvr, boutr, out_ref):
    H = h_ref[0]
    cur = nf_ref[0]
    ndeg = nd_ref[0]
    Wn = (wn0[...], wn1[...], wn2[...], wn3[...])
    Bn = (bn0[...], bn1[...], bn2[...], bn3[...])
    We = (we0[...], we1[...], we2[...], we3[...])
    Be = (be0[...], be1[...], be2[...], be3[...])

    cats = []
    for lv in range(4):
        P = _mm(cur, Wn[lv])
        n2e = _mm(H, P)
        msg = jnp.tanh(n2e + Bn[lv])
        if lv < 3:
            Q = _mm(msg, We[lv])
        else:
            Q = msg * We[lv][0, 0]
        e2n = _mmT(H, Q)
        cur = jnp.tanh((e2n + Be[lv]) / ndeg)
        cats.append(cur)

    v = cats[3]
    ii = lax.broadcasted_iota(jnp.int32, (_NG, _NG), 0)
    jj = lax.broadcasted_iota(jnp.int32, (_NG, _NG), 1)
    eye = (ii == jj).astype(jnp.float32)
    vrow = _mmT(v, eye)
    before = (vrow > v) | ((vrow == v) & (jj < ii))
    r = jnp.sum(before.astype(jnp.float32), axis=1, keepdims=True)
    ks = lax.broadcasted_iota(jnp.int32, (_NG, _K), 1).astype(jnp.float32)
    O = (r == ks).astype(jnp.float32)

    W1T = w1t[...]
    y1 = (_mm(_mmT(O, cats[0]), W1T[0:32])
          + _mm(_mmT(O, cats[1]), W1T[32:64])
          + _mm(_mmT(O, cats[2]), W1T[64:96])
          + _mmT(O, cats[3]) * W1T[96:97]
          + b1r[...])
    y1 = jnp.maximum(y1, 0.0)

    ri = lax.broadcasted_iota(jnp.int32, (_K // 2, _K), 0)
    ci = lax.broadcasted_iota(jnp.int32, (_K // 2, _K), 1)
    ev = (ci == 2 * ri).astype(jnp.float32)
    od = (ci == 2 * ri + 1).astype(jnp.float32)
    y1p = jnp.maximum(_mm(ev, y1), _mm(od, y1))

    W2r = w2r[...]
    acc = jnp.zeros((11, 32), jnp.float32) + b2r[...]
    for t in range(5):
        acc = acc + _mm(lax.slice(y1p, (t, 0), (t + 11, 16)), W2r[t])
    y2 = jnp.maximum(acc, 0.0)

    Wv = wvr[...]
    o = jnp.zeros((1, 128), jnp.float32) + boutr[...]
    for h in range(11):
        o = o + _mm(lax.slice(y2, (h, 0), (h + 1, 32)),
                    lax.slice(Wv, (h * 32, 0), (h * 32 + 32, 128)))
    out_ref[0] = jnp.maximum(o, 0.0)


def _tc_forward(Hg, nf, nd, Wn, bn, We, be, W1T, b1r, W2r, b2r, Wv, boutr,
                interpret=False):
    def full(x):
        return pl.BlockSpec(x.shape, lambda i: (0,) * x.ndim)

    weights = (*Wn, *bn, *We, *be, W1T, b1r, W2r, b2r, Wv, boutr)
    return pl.pallas_call(
        _tc_body,
        grid=(_B,),
        in_specs=[
            pl.BlockSpec((1, _MG, _NG), lambda i: (i, 0, 0)),
            pl.BlockSpec((1, _NG, _D), lambda i: (i, 0, 0)),
            pl.BlockSpec((1, _NG, 1), lambda i: (i, 0, 0)),
            *[full(w) for w in weights],
        ],
        out_specs=pl.BlockSpec((1, 1, 128), lambda i: (i, 0, 0)),
        out_shape=jax.ShapeDtypeStruct((_B, 1, 128), jnp.float32),
        interpret=interpret,
    )(Hg, nf, nd, *weights).reshape(_B, 128)


def kernel(node_feat, edge_index, inc_vals, node_degs, edge_degs,
           Wn, bn, We, be, W1, b1, W2, b2, Wout, bout):
    e_ids = edge_index[0].astype(jnp.int32)
    n_ids = edge_index[1].astype(jnp.int32)

    hflat = _get_build_h()(e_ids, n_ids, inc_vals.astype(jnp.float32))
    Hg = hflat.reshape(_B, _MG, _NG)

    nf = node_feat.reshape(_B, _NG, _D)
    nd = node_degs.reshape(_B, _NG, 1)
    W1T = W1.T
    W2r = jnp.transpose(W2, (2, 1, 0))
    Wv = jnp.transpose(Wout.reshape(32, 11, 128), (1, 0, 2)).reshape(352, 128)
    bn2 = tuple(b.reshape(1, -1) for b in bn)
    be2 = tuple(b.reshape(1, -1) for b in be)
    return _tc_forward(Hg, nf, nd, Wn, bn2, We, be2,
                       W1T, b1.reshape(1, -1), W2r, b2.reshape(1, -1),
                       Wv, bout.reshape(1, -1))
